# baseline (speedup 1.0000x reference)
import functools

import jax
import jax.numpy as jnp
from jax import lax
from jax.experimental import pallas as pl
from jax.experimental.pallas import tpu as pltpu
from jax.experimental.pallas import tpu_sc as plsc

SEQ_LEN = 8192
D_MODEL = 768
BATCH = 4
EPS = 1e-12

NC = 2
NS = 16
NW = NC * NS
SEQ_PER_W = SEQ_LEN // NW
CHUNK = 64
N_CHUNKS = SEQ_PER_W // CHUNK
GROUPS = CHUNK // 16


def _rsqrt_newton(v):
    i = plsc.bitcast(v, jnp.int32)
    y = plsc.bitcast(jnp.int32(0x5F3759DF) - (i >> 1), jnp.float32)
    h = v * 0.5
    for _ in range(3):
        y = y * (1.5 - h * y * y)
    return y


def _sc_body(in_hbm, pos_hbm, gamma_hbm, beta_hbm, out_hbm,
             x_v, pos_v, xt_v, gamma_v, beta_v):
    wid = lax.axis_index("s") * NC + lax.axis_index("c")
    base_seq = wid * SEQ_PER_W
    pltpu.sync_copy(gamma_hbm, gamma_v)
    pltpu.sync_copy(beta_hbm, beta_v)
    lane = lax.iota(jnp.int32, 16)
    zero16 = jnp.zeros((16,), jnp.float32)
    inv_d = 1.0 / D_MODEL

    def chunk_body(c, _):
        seq0 = base_seq + c * CHUNK
        pltpu.sync_copy(pos_hbm.at[pl.ds(seq0, CHUNK)], pos_v)

        def batch_body(b, _):
            pltpu.sync_copy(in_hbm.at[b, pl.ds(seq0, CHUNK)], x_v)

            def group_body(g, _):
                ridx = lane + g * 16

                def p1(j, carry, ridx=ridx):
                    s, s2 = carry
                    cidx = jnp.full((16,), j, jnp.int32)
                    a = plsc.load_gather(x_v, [ridx, cidx])
                    p = plsc.load_gather(pos_v, [ridx, cidx])
                    x = a + p
                    xt_v[j] = x
                    return (s + x, s2 + x * x)

                s, s2 = lax.fori_loop(0, D_MODEL, p1, (zero16, zero16),
                                      unroll=8)
                mean16 = s * inv_d
                var16 = s2 * inv_d - mean16 * mean16
                scale16 = _rsqrt_newton(var16 + EPS)
                shift16 = -mean16 * scale16

                def p2(jj, carry, ridx=ridx, scale16=scale16,
                       shift16=shift16):
                    gvec = gamma_v[pl.ds(jj * 16, 16)]
                    bvec = beta_v[pl.ds(jj * 16, 16)]
                    for k in range(16):
                        j = jj * 16 + k
                        cidx = jnp.full((16,), j, jnp.int32)
                        x = xt_v[j]
                        t = x * scale16 + shift16
                        o = t * gvec[k] + bvec[k]
                        plsc.store_scatter(x_v, [ridx, cidx], o)
                    return carry

                lax.fori_loop(0, D_MODEL // 16, p2, 0)
                return 0

            lax.fori_loop(0, GROUPS, group_body, 0)
            pltpu.sync_copy(x_v, out_hbm.at[b, pl.ds(seq0, CHUNK)])
            return 0

        lax.fori_loop(0, BATCH, batch_body, 0)
        return 0

    lax.fori_loop(0, N_CHUNKS, chunk_body, 0)


@jax.jit
def kernel(inputs_embeds, pos_table, ln_gamma, ln_beta):
    mesh = plsc.VectorSubcoreMesh(core_axis_name="c", subcore_axis_name="s")
    f = pl.kernel(
        _sc_body,
        out_type=jax.ShapeDtypeStruct((BATCH, SEQ_LEN, D_MODEL), jnp.float32),
        mesh=mesh,
        scratch_types=[
            pltpu.VMEM((CHUNK, D_MODEL), jnp.float32),
            pltpu.VMEM((CHUNK, D_MODEL), jnp.float32),
            pltpu.VMEM((D_MODEL, 16), jnp.float32),
            pltpu.VMEM((D_MODEL,), jnp.float32),
            pltpu.VMEM((D_MODEL,), jnp.float32),
        ],
        compiler_params=pltpu.CompilerParams(
            use_tc_tiling_on_sc=False, needs_layout_passes=False),
    )
    return f(inputs_embeds, pos_table, ln_gamma, ln_beta)


# speedup vs baseline: 1.0348x; 1.0348x over previous
import functools

import jax
import jax.numpy as jnp
from jax import lax
from jax.experimental import pallas as pl
from jax.experimental.pallas import tpu as pltpu
from jax.experimental.pallas import tpu_sc as plsc

SEQ_LEN = 8192
D_MODEL = 768
BATCH = 4
EPS = 1e-12

NC = 2
NS = 16
NW = NC * NS
SEQ_PER_W = SEQ_LEN // NW
CHUNK = 64
N_CHUNKS = SEQ_PER_W // CHUNK
GROUPS = CHUNK // 16
CW = CHUNK * D_MODEL


def _rsqrt_newton(v):
    i = plsc.bitcast(v, jnp.int32)
    y = plsc.bitcast(jnp.int32(0x5F3759DF) - (i >> 1), jnp.float32)
    h = v * 0.5
    for _ in range(3):
        y = y * (1.5 - h * y * y)
    return y


def _sc_body(in_hbm, pos_hbm, gamma_hbm, beta_hbm, out_hbm,
             x_v, pos_v, xt_v, gamma_v, beta_v):
    wid = lax.axis_index("s") * NC + lax.axis_index("c")
    base_seq = wid * SEQ_PER_W
    pltpu.sync_copy(gamma_hbm, gamma_v)
    pltpu.sync_copy(beta_hbm, beta_v)
    lane = lax.iota(jnp.int32, 16)
    zero16 = jnp.zeros((16,), jnp.float32)
    one16 = jnp.ones((16,), jnp.int32)
    inv_d = 1.0 / D_MODEL

    def chunk_body(c, _):
        seq0 = base_seq + c * CHUNK
        pltpu.sync_copy(pos_hbm.at[pl.ds(seq0 * D_MODEL, CW)], pos_v)

        def batch_body(b, _):
            off = (b * SEQ_LEN + seq0) * D_MODEL
            pltpu.sync_copy(in_hbm.at[pl.ds(off, CW)], x_v)

            def group_body(g, _):
                idx0 = (lane + g * 16) * D_MODEL

                def p1(j, carry):
                    s, s2, idx = carry
                    a = plsc.load_gather(x_v, [idx])
                    p = plsc.load_gather(pos_v, [idx])
                    x = a + p
                    xt_v[pl.ds(j * 16, 16)] = x
                    return (s + x, s2 + x * x, idx + one16)

                s, s2, _unused = lax.fori_loop(
                    0, D_MODEL, p1, (zero16, zero16, idx0), unroll=16)
                mean16 = s * inv_d
                var16 = s2 * inv_d - mean16 * mean16
                scale16 = _rsqrt_newton(var16 + EPS)
                shift16 = -mean16 * scale16

                def p2(jj, idx, scale16=scale16, shift16=shift16):
                    gvec = gamma_v[pl.ds(jj * 16, 16)]
                    bvec = beta_v[pl.ds(jj * 16, 16)]
                    for k in range(16):
                        x = xt_v[pl.ds(jj * 256 + k * 16, 16)]
                        t = x * scale16 + shift16
                        o = t * gvec[k] + bvec[k]
                        plsc.store_scatter(x_v, [idx], o)
                        idx = idx + one16
                    return idx

                lax.fori_loop(0, D_MODEL // 16, p2, idx0)
                return 0

            lax.fori_loop(0, GROUPS, group_body, 0)
            pltpu.sync_copy(x_v, out_hbm.at[pl.ds(off, CW)])
            return 0

        lax.fori_loop(0, BATCH, batch_body, 0)
        return 0

    lax.fori_loop(0, N_CHUNKS, chunk_body, 0)


@jax.jit
def kernel(inputs_embeds, pos_table, ln_gamma, ln_beta):
    mesh = plsc.VectorSubcoreMesh(core_axis_name="c", subcore_axis_name="s")
    f = pl.kernel(
        _sc_body,
        out_type=jax.ShapeDtypeStruct((BATCH * SEQ_LEN * D_MODEL,),
                                      jnp.float32),
        mesh=mesh,
        scratch_types=[
            pltpu.VMEM((CW,), jnp.float32),
            pltpu.VMEM((CW,), jnp.float32),
            pltpu.VMEM((16 * D_MODEL,), jnp.float32),
            pltpu.VMEM((D_MODEL,), jnp.float32),
            pltpu.VMEM((D_MODEL,), jnp.float32),
        ],
        compiler_params=pltpu.CompilerParams(
            use_tc_tiling_on_sc=False, needs_layout_passes=False),
    )
    out = f(inputs_embeds.reshape(-1), pos_table.reshape(-1),
            ln_gamma, ln_beta)
    return out.reshape(BATCH, SEQ_LEN, D_MODEL)


# speedup vs baseline: 27.3309x; 26.4116x over previous
import functools

import jax
import jax.numpy as jnp
from jax.experimental import pallas as pl
from jax.experimental.pallas import tpu as pltpu

SEQ_LEN = 8192
D_MODEL = 768
BATCH = 4
EPS = 1e-12

BLOCK_ROWS = 2048


def _ln_kernel(x_ref, pos_ref, gamma_ref, beta_ref, out_ref):
    x = x_ref[0] + pos_ref[...]
    inv_d = 1.0 / D_MODEL
    m = jnp.sum(x, axis=-1, keepdims=True) * inv_d
    m2 = jnp.sum(x * x, axis=-1, keepdims=True) * inv_d
    var = m2 - m * m
    rs = jax.lax.rsqrt(var + EPS)
    c = -m * rs
    t = x * rs + c
    out_ref[0] = t * gamma_ref[...] + beta_ref[...]


@jax.jit
def kernel(inputs_embeds, pos_table, ln_gamma, ln_beta):
    num_seq_blocks = SEQ_LEN // BLOCK_ROWS
    grid = (num_seq_blocks, BATCH)
    return pl.pallas_call(
        _ln_kernel,
        grid=grid,
        in_specs=[
            pl.BlockSpec((1, BLOCK_ROWS, D_MODEL), lambda i, j: (j, i, 0)),
            pl.BlockSpec((BLOCK_ROWS, D_MODEL), lambda i, j: (i, 0)),
            pl.BlockSpec((D_MODEL,), lambda i, j: (0,)),
            pl.BlockSpec((D_MODEL,), lambda i, j: (0,)),
        ],
        out_specs=pl.BlockSpec((1, BLOCK_ROWS, D_MODEL), lambda i, j: (j, i, 0)),
        out_shape=jax.ShapeDtypeStruct((BATCH, SEQ_LEN, D_MODEL), jnp.float32),
        compiler_params=pltpu.CompilerParams(
            dimension_semantics=("arbitrary", "arbitrary"),
        ),
    )(inputs_embeds, pos_table, ln_gamma, ln_beta)


# speedup vs baseline: 28.8486x; 1.0555x over previous
import jax
import jax.numpy as jnp
from jax.experimental import pallas as pl
from jax.experimental.pallas import tpu as pltpu

SEQ_LEN = 8192
D_MODEL = 768
BATCH = 4
EPS = 1e-12

BLOCK_ROWS = 2048


def _ln_kernel(x_ref, pos_ref, gamma_ref, beta_ref, out_ref):
    i = pl.program_id(1)
    x = x_ref[0] + pos_ref[pl.ds(i * BLOCK_ROWS, BLOCK_ROWS), :]
    inv_d = 1.0 / D_MODEL
    m = jnp.sum(x, axis=-1, keepdims=True) * inv_d
    m2 = jnp.sum(x * x, axis=-1, keepdims=True) * inv_d
    var = m2 - m * m
    rs = jax.lax.rsqrt(var + EPS)
    c = -m * rs
    t = x * rs + c
    out_ref[0] = t * gamma_ref[...] + beta_ref[...]


@jax.jit
def kernel(inputs_embeds, pos_table, ln_gamma, ln_beta):
    num_seq_blocks = SEQ_LEN // BLOCK_ROWS
    grid = (BATCH, num_seq_blocks)
    return pl.pallas_call(
        _ln_kernel,
        grid=grid,
        in_specs=[
            pl.BlockSpec((1, BLOCK_ROWS, D_MODEL), lambda j, i: (j, i, 0)),
            pl.BlockSpec((SEQ_LEN, D_MODEL), lambda j, i: (0, 0)),
            pl.BlockSpec((D_MODEL,), lambda j, i: (0,)),
            pl.BlockSpec((D_MODEL,), lambda j, i: (0,)),
        ],
        out_specs=pl.BlockSpec((1, BLOCK_ROWS, D_MODEL), lambda j, i: (j, i, 0)),
        out_shape=jax.ShapeDtypeStruct((BATCH, SEQ_LEN, D_MODEL), jnp.float32),
        compiler_params=pltpu.CompilerParams(
            dimension_semantics=("arbitrary", "arbitrary"),
        ),
    )(inputs_embeds, pos_table, ln_gamma, ln_beta)
